# Initial kernel scaffold; baseline (speedup 1.0000x reference)
import jax
import jax.numpy as jnp
from jax.experimental import pallas as pl


def kernel(idx, val, labels, buffer_imgs, buffer_labels):
    raise NotImplementedError("write your pallas kernel here")


# speedup vs baseline: 9.3217x; 9.3217x over previous
import jax
import jax.numpy as jnp
from jax import lax
from jax.experimental import pallas as pl
from jax.experimental.pallas import tpu as pltpu
from jax.experimental.pallas import tpu_sc as plsc

_B = 16384
_D = 64
_L = 16
_NC = 2
_NS = 16
_SHIFT = 15
_WSZ = 1 << _SHIFT
_CH = 128
_NCHUNK = _B // _CH

_SENT = 0x7FFFFFFF


def _shift_up(x):
    i = jnp.minimum(lax.iota(jnp.int32, _L) + 1, _L - 1)
    dnums = lax.GatherDimensionNumbers(
        offset_dims=(), collapsed_slice_dims=(0,), start_index_map=(0,))
    return lax.gather(x, i[:, None], dnums, (1,),
                      mode=lax.GatherScatterMode.PROMISE_IN_BOUNDS)


def _body(idx_hbm, val_hbm, lab_hbm, out_hbm, outlab_hbm,
          idx_v, W, my_i, my_loc, rows, lrow, sem):
    wid = lax.axis_index("s") * _NC + lax.axis_index("c")
    pltpu.sync_copy(idx_hbm, idx_v)
    lane = lax.iota(jnp.int32, _L)

    def p1(k, cnt):
        v = idx_v[pl.ds(k * _L, _L)]
        owner = lax.shift_right_logical(v, _SHIFT)
        m = owner == wid
        local = lax.bitwise_and(v, _WSZ - 1)
        key = jnp.where(m, lax.bitwise_or(lax.shift_left(local, 4), lane),
                        _SENT)
        skey, _ = plsc.sort_key_val(key, key)
        sloc_cmp = lax.shift_right_logical(skey, 4)
        keep = (sloc_cmp != _shift_up(sloc_cmp)) | (lane == _L - 1)
        wr = keep & (skey != _SENT)
        sloc = lax.bitwise_and(sloc_cmp, _WSZ - 1)
        j_sorted = k * _L + lax.bitwise_and(skey, _L - 1)
        plsc.store_scatter(W, [sloc], j_sorted, mask=wr)
        cum = plsc.cumsum(m.astype(jnp.int32))
        addr = cnt + cum - 1
        hi = lax.shift_right_logical(addr, 7)
        lo = lax.bitwise_and(addr, _CH - 1)
        plsc.store_scatter(my_i, [hi, lo], k * _L + lane, mask=m)
        plsc.store_scatter(my_loc, [hi, lo], local, mask=m)
        return cnt + jnp.sum(m.astype(jnp.int32))

    cnt = lax.fori_loop(0, _B // _L, p1, jnp.int32(0))

    nchunks = lax.shift_right_logical(cnt + _CH - 1, 7)
    zero16 = jnp.zeros((_L,), jnp.int32)
    i0 = plsc.load_gather(my_i, [zero16, zero16])
    l0 = lax.bitwise_and(plsc.load_gather(my_loc, [zero16, zero16]), _WSZ - 1)
    r0 = plsc.load_gather(W, [l0])

    def p2a(t, _):
        pos = t * _L + lane
        hi = lax.shift_right_logical(pos, 7)
        lo = lax.bitwise_and(pos, _CH - 1)
        valid = pos < cnt
        loc = lax.bitwise_and(plsc.load_gather(my_loc, [hi, lo]), _WSZ - 1)
        r = jnp.where(valid, plsc.load_gather(W, [loc]), r0)
        iv = jnp.where(valid, plsc.load_gather(my_i, [hi, lo]), i0)
        plsc.store_scatter(my_loc, [hi, lo], r)
        plsc.store_scatter(my_i, [hi, lo], iv)
        return 0

    lax.fori_loop(0, nchunks * (_CH // _L), p2a, 0)

    def p2b(c, _):
        r_row = my_loc.at[c]
        i_row = my_i.at[c]
        pltpu.async_copy(val_hbm.at[r_row], rows, sem).wait()
        pltpu.async_copy(rows, out_hbm.at[i_row], sem).wait()
        pltpu.async_copy(lab_hbm.at[r_row], lrow, sem).wait()
        pltpu.async_copy(lrow, outlab_hbm.at[i_row], sem).wait()
        return 0

    lax.fori_loop(0, nchunks, p2b, 0)


def kernel(idx, val, labels, buffer_imgs, buffer_labels):
    del buffer_imgs, buffer_labels
    f = pl.kernel(
        _body,
        out_type=(
            jax.ShapeDtypeStruct((_B, _D), jnp.float32),
            jax.ShapeDtypeStruct((_B,), jnp.int32),
        ),
        mesh=plsc.VectorSubcoreMesh(core_axis_name="c", subcore_axis_name="s"),
        compiler_params=pltpu.CompilerParams(
            needs_layout_passes=False, use_tc_tiling_on_sc=False),
        scratch_types=[
            pltpu.VMEM((_B,), jnp.int32),
            pltpu.VMEM((_WSZ,), jnp.int32),
            pltpu.VMEM((_NCHUNK, _CH), jnp.int32),
            pltpu.VMEM((_NCHUNK, _CH), jnp.int32),
            pltpu.VMEM((_CH, _D), jnp.float32),
            pltpu.VMEM((_CH,), jnp.int32),
            pltpu.SemaphoreType.DMA,
        ],
    )
    return f(idx.astype(jnp.int32), val, labels.astype(jnp.int32))
